# baseline (speedup 1.0000x reference)
import functools

import jax
import jax.numpy as jnp
from jax import lax
from jax.experimental import pallas as pl
from jax.experimental.pallas import tpu as pltpu
from jax.experimental.pallas import tpu_sc as plsc

NUM_OUTPUTS = 1000
ROWS = 4096
COLS = 26
FLAT = ROWS * COLS

NW = 32
ROWS_PW = ROWS // NW
IDX_PW = ROWS_PW * COLS
BATCH_R = 4
PAIRS = BATCH_R * COLS
NBATCH = ROWS_PW // BATCH_R
GROUP_OFF = [0, 16, 32, 48, 64, 80, PAIRS - 16]
NGROUP = len(GROUP_OFF)

_mesh = plsc.VectorSubcoreMesh(core_axis_name="c", subcore_axis_name="s")


@functools.partial(
    pl.kernel,
    mesh=_mesh,
    out_type=jax.ShapeDtypeStruct((ROWS, COLS, NUM_OUTPUTS), jnp.int32),
    scratch_types=[
        pltpu.VMEM((IDX_PW,), jnp.int32),
        pltpu.VMEM((BATCH_R, COLS, NUM_OUTPUTS), jnp.int32),
        pltpu.VMEM((NGROUP * 16,), jnp.int32),
    ],
    compiler_params=pltpu.CompilerParams(
        needs_layout_passes=False, use_tc_tiling_on_sc=False
    ),
)
def _sc_onehot(idx_hbm, out_hbm, idx_v, buf, kv_save):
    wid = lax.axis_index("s") * 2 + lax.axis_index("c")
    row0 = wid * ROWS_PW

    zeros16 = jnp.zeros((16,), jnp.int32)
    ones16 = zeros16 + 1
    lane = lax.iota(jnp.int32, 16)

    rv_g, cv_g = [], []
    for g in range(NGROUP):
        p = lane + GROUP_OFF[g]
        rv_g.append(p // COLS)
        cv_g.append(p % COLS)

    pltpu.sync_copy(idx_hbm.at[pl.ds(wid * IDX_PW, IDX_PW)], idx_v)

    def _zero_pair(p, carry):
        rv = p // COLS
        cv = p % COLS

        def _zero_k(j, c):
            buf[rv, cv, pl.ds(j * 16, 16)] = zeros16
            return c

        lax.fori_loop(0, NUM_OUTPUTS // 16, _zero_k, 0)
        buf[rv, cv, pl.ds(NUM_OUTPUTS - 16, 16)] = zeros16
        return carry

    lax.fori_loop(0, PAIRS, _zero_pair, 0)

    for g in range(NGROUP):
        kv_save[pl.ds(g * 16, 16)] = zeros16

    def _batch(t, carry):
        for g in range(NGROUP):
            old = kv_save[pl.ds(g * 16, 16)]
            plsc.store_scatter(buf, [rv_g[g], cv_g[g], old], zeros16)
        for g in range(NGROUP):
            kv = idx_v[pl.ds(t * PAIRS + GROUP_OFF[g], 16)]
            plsc.store_scatter(buf, [rv_g[g], cv_g[g], kv], ones16)
            kv_save[pl.ds(g * 16, 16)] = kv
        pltpu.sync_copy(buf, out_hbm.at[pl.ds(row0 + t * BATCH_R, BATCH_R)])
        return carry

    lax.fori_loop(0, NBATCH, _batch, 0)


def kernel(inputs):
    return _sc_onehot(inputs.reshape(FLAT))


# speedup vs baseline: 1.9905x; 1.9905x over previous
import functools

import jax
import jax.numpy as jnp
from jax import lax
from jax.experimental import pallas as pl
from jax.experimental.pallas import tpu as pltpu
from jax.experimental.pallas import tpu_sc as plsc

NUM_OUTPUTS = 1000
ROWS = 4096
COLS = 26
FLAT = ROWS * COLS

NW = 32
ROWS_PW = ROWS // NW
IDX_PW = ROWS_PW * COLS
BATCH_R = 2
PAIRS = BATCH_R * COLS
NBATCH = ROWS_PW // BATCH_R
GROUP_OFF = [0, 16, 32, PAIRS - 16]
NGROUP = len(GROUP_OFF)

_mesh = plsc.VectorSubcoreMesh(core_axis_name="c", subcore_axis_name="s")


@functools.partial(
    pl.kernel,
    mesh=_mesh,
    out_type=jax.ShapeDtypeStruct((ROWS, COLS, NUM_OUTPUTS), jnp.int32),
    scratch_types=[
        pltpu.VMEM((IDX_PW,), jnp.int32),
        pltpu.VMEM((BATCH_R, COLS, NUM_OUTPUTS), jnp.int32),
        pltpu.VMEM((NGROUP * 16,), jnp.int32),
    ],
    compiler_params=pltpu.CompilerParams(
        needs_layout_passes=False, use_tc_tiling_on_sc=True
    ),
)
def _sc_onehot(idx_hbm, out_hbm, idx_v, buf, kv_save):
    wid = lax.axis_index("s") * 2 + lax.axis_index("c")
    row0 = wid * ROWS_PW

    zeros16 = jnp.zeros((16,), jnp.int32)
    ones16 = zeros16 + 1
    lane = lax.iota(jnp.int32, 16)

    rv_g, cv_g = [], []
    for g in range(NGROUP):
        p = lane + GROUP_OFF[g]
        rv_g.append(p // COLS)
        cv_g.append(p % COLS)

    pltpu.sync_copy(idx_hbm.at[pl.ds(wid * IDX_PW, IDX_PW)], idx_v)

    def _zero_pair(p, carry):
        rv = p // COLS
        cv = p % COLS

        def _zero_k(j, c):
            buf[rv, cv, pl.ds(j * 16, 16)] = zeros16
            return c

        lax.fori_loop(0, NUM_OUTPUTS // 16, _zero_k, 0)
        buf[rv, cv, pl.ds(NUM_OUTPUTS - 16, 16)] = zeros16
        return carry

    lax.fori_loop(0, PAIRS, _zero_pair, 0)

    for g in range(NGROUP):
        kv_save[pl.ds(g * 16, 16)] = zeros16

    def _batch(t, carry):
        for g in range(NGROUP):
            old = kv_save[pl.ds(g * 16, 16)]
            plsc.store_scatter(buf, [rv_g[g], cv_g[g], old], zeros16)
        for g in range(NGROUP):
            kv = idx_v[pl.ds(t * PAIRS + GROUP_OFF[g], 16)]
            plsc.store_scatter(buf, [rv_g[g], cv_g[g], kv], ones16)
            kv_save[pl.ds(g * 16, 16)] = kv
        pltpu.sync_copy(buf, out_hbm.at[pl.ds(row0 + t * BATCH_R, BATCH_R)])
        return carry

    lax.fori_loop(0, NBATCH, _batch, 0)


def kernel(inputs):
    return _sc_onehot(inputs.reshape(FLAT))


# speedup vs baseline: 1.9953x; 1.0024x over previous
import functools

import jax
import jax.numpy as jnp
from jax import lax
from jax.experimental import pallas as pl
from jax.experimental.pallas import tpu as pltpu
from jax.experimental.pallas import tpu_sc as plsc

NUM_OUTPUTS = 1000
ROWS = 4096
COLS = 26
FLAT = ROWS * COLS

NW = 32
ROWS_PW = ROWS // NW
IDX_PW = ROWS_PW * COLS
GROUP_OFF = [0, COLS - 16]
NGROUP = len(GROUP_OFF)
NBUF = 2

_mesh = plsc.VectorSubcoreMesh(core_axis_name="c", subcore_axis_name="s")


@functools.partial(
    pl.kernel,
    mesh=_mesh,
    out_type=jax.ShapeDtypeStruct((ROWS, COLS, NUM_OUTPUTS), jnp.int32),
    scratch_types=[
        pltpu.VMEM((IDX_PW,), jnp.int32),
        pltpu.VMEM((1, COLS, NUM_OUTPUTS), jnp.int32),
        pltpu.VMEM((1, COLS, NUM_OUTPUTS), jnp.int32),
        pltpu.VMEM((NBUF * NGROUP * 16,), jnp.int32),
        pltpu.SemaphoreType.DMA,
        pltpu.SemaphoreType.DMA,
    ],
    compiler_params=pltpu.CompilerParams(
        needs_layout_passes=False, use_tc_tiling_on_sc=True
    ),
)
def _sc_onehot(idx_hbm, out_hbm, idx_v, buf0, buf1, kv_save, sem0, sem1):
    wid = lax.axis_index("s") * 2 + lax.axis_index("c")
    row0 = wid * ROWS_PW

    bufs = (buf0, buf1)
    sems = (sem0, sem1)
    zeros16 = jnp.zeros((16,), jnp.int32)
    ones16 = zeros16 + 1
    lane = lax.iota(jnp.int32, 16)
    rv16 = lane * 0
    cv_g = [jnp.minimum(lane + off, COLS - 1) for off in GROUP_OFF]

    pltpu.sync_copy(idx_hbm.at[pl.ds(wid * IDX_PW, IDX_PW)], idx_v)

    def _zero_col(p, carry):
        def _zero_k(j, c):
            for b in range(NBUF):
                bufs[b][0, p, pl.ds(j * 16, 16)] = zeros16
            return c

        lax.fori_loop(0, NUM_OUTPUTS // 16, _zero_k, 0)
        for b in range(NBUF):
            bufs[b][0, p, pl.ds(NUM_OUTPUTS - 16, 16)] = zeros16
        return carry

    lax.fori_loop(0, COLS, _zero_col, 0)

    for g in range(NBUF * NGROUP):
        kv_save[pl.ds(g * 16, 16)] = zeros16

    def _row(t, b):
        for g in range(NGROUP):
            old = kv_save[pl.ds((b * NGROUP + g) * 16, 16)]
            plsc.store_scatter(bufs[b], [rv16, cv_g[g], old], zeros16)
        for g in range(NGROUP):
            kv = idx_v[pl.ds(t * COLS + GROUP_OFF[g], 16)]
            plsc.store_scatter(bufs[b], [rv16, cv_g[g], kv], ones16)
            kv_save[pl.ds((b * NGROUP + g) * 16, 16)] = kv
        pltpu.async_copy(bufs[b], out_hbm.at[pl.ds(row0 + t, 1)], sems[b])

    for b in range(NBUF):
        _row(b, b)

    def _steady(o, carry):
        for b in range(NBUF):
            t = o * NBUF + b
            pltpu.make_async_copy(
                bufs[b], out_hbm.at[pl.ds(row0 + t, 1)], sems[b]
            ).wait()
            _row(t, b)
        return carry

    lax.fori_loop(1, ROWS_PW // NBUF, _steady, 0)

    for b in range(NBUF):
        pltpu.make_async_copy(
            bufs[b], out_hbm.at[pl.ds(row0 + b, 1)], sems[b]
        ).wait()


def kernel(inputs):
    return _sc_onehot(inputs.reshape(FLAT))
